# Initial kernel scaffold; baseline (speedup 1.0000x reference)
import jax
import jax.numpy as jnp
from jax.experimental import pallas as pl


def kernel(x, edge_index, W1, a_src1, a_dst1, b1, W_mu, a_src_mu, a_dst_mu, b_mu, W_ls, a_src_ls, a_dst_ls, b_ls):
    raise NotImplementedError("write your pallas kernel here")


# speedup vs baseline: 14.4695x; 14.4695x over previous
import functools

import jax
import jax.numpy as jnp
from jax import lax
from jax.experimental import pallas as pl
from jax.experimental.pallas import tpu as pltpu
from jax.experimental.pallas import tpu_sc as plsc

N_NODES = 10000
N_TILES = 32


def _make_edge_pass(E, C, heads):
    OUTW = C + 16
    e_per_tile = E // N_TILES
    K = 80
    nchunk = e_per_tile // K
    accn = 10240
    rows_per_tile = accn // 16
    zrows = 128
    mesh = plsc.VectorSubcoreMesh(core_axis_name="c", subcore_axis_name="s")

    @functools.partial(
        pl.kernel,
        mesh=mesh,
        compiler_params=pltpu.CompilerParams(
            needs_layout_passes=False, use_tc_tiling_on_sc=False),
        out_type=jax.ShapeDtypeStruct((2, heads, accn, OUTW), jnp.float32),
        scratch_types=[
            pltpu.VMEM((K,), jnp.int32),
            pltpu.VMEM((K,), jnp.int32),
            pltpu.VMEM((K,), jnp.int32),
            pltpu.VMEM((K,), jnp.float32),
            pltpu.VMEM((N_NODES,), jnp.float32),
            pltpu.VMEM((N_NODES,), jnp.float32),
            pltpu.VMEM((heads, 16), jnp.float32),
            pltpu.VMEM((K, C), jnp.float32),
            pltpu.VMEM((K, OUTW), jnp.float32),
            pltpu.VMEM((zrows, OUTW), jnp.float32),
            pltpu.VMEM_SHARED((accn, OUTW), jnp.float32),
            pltpu.SemaphoreType.DMA,
        ],
    )
    def edge_pass(src_hbm, dst_hbm, htab_hbm, as_hbm, ad_hbm, m_hbm, out_hbm,
                  src_v, dst_v, srcp_v, w_v, as_v, ad_v, m_v,
                  rin_v, rout_v, zero_v, acc_sh, sem):
        cid = lax.axis_index("c")
        sid = lax.axis_index("s")
        zvec = jnp.zeros((16,), jnp.float32)
        lane0 = jnp.arange(16, dtype=jnp.int32) == 0

        def zfill(i, carry):
            for j in range(OUTW // 16):
                zero_v[i, pl.ds(j * 16, 16)] = zvec
            return carry

        lax.fori_loop(0, zrows, zfill, 0)
        pltpu.sync_copy(m_hbm, m_v)
        row_base = sid * rows_per_tile
        ebase = cid * (E // 2) + sid * e_per_tile

        for hd in range(heads):
            pltpu.sync_copy(as_hbm.at[pl.ds(hd * N_NODES, N_NODES)], as_v)
            pltpu.sync_copy(ad_hbm.at[pl.ds(hd * N_NODES, N_NODES)], ad_v)

            def zero_acc(i, carry):
                pltpu.sync_copy(zero_v, acc_sh.at[pl.ds(row_base + i * zrows, zrows)])
                return carry

            lax.fori_loop(0, rows_per_tile // zrows, zero_acc, 0)
            plsc.subcore_barrier()

            mvec = m_v[hd, :]

            def chunk(i, carry):
                cb = ebase + i * K
                pltpu.sync_copy(src_hbm.at[pl.ds(cb, K)], src_v)
                pltpu.sync_copy(dst_hbm.at[pl.ds(cb, K)], dst_v)
                for j in range(K // 16):
                    srcp_v[pl.ds(j * 16, 16)] = (
                        src_v[pl.ds(j * 16, 16)] + hd * N_NODES)
                pltpu.async_copy(htab_hbm.at[srcp_v], rin_v, sem).wait()
                for j in range(K // 16):
                    s16 = src_v[pl.ds(j * 16, 16)]
                    d16 = dst_v[pl.ds(j * 16, 16)]
                    z = (plsc.load_gather(as_v, [s16])
                         + plsc.load_gather(ad_v, [d16]))
                    alpha = jnp.where(z > 0, z, z * 0.2)
                    w_v[pl.ds(j * 16, 16)] = jnp.exp(alpha - mvec)

                def edge_grp(g, c2):
                    w16 = w_v[pl.ds(g * 16, 16)]
                    for l in range(16):
                        ws = w16[l]
                        e = g * 16 + l
                        for q in range(C // 16):
                            rout_v[e, pl.ds(q * 16, 16)] = (
                                rin_v[e, pl.ds(q * 16, 16)] * ws)
                        rout_v[e, pl.ds(C, 16)] = jnp.where(lane0, ws, 0.0)
                    return c2

                lax.fori_loop(0, K // 16, edge_grp, 0)
                pltpu.sync_copy(rout_v, acc_sh.at[dst_v], add=True)
                return carry

            lax.fori_loop(0, nchunk, chunk, 0)
            plsc.subcore_barrier()

            def copy_out(i, carry):
                r0 = row_base + i * zrows
                pltpu.sync_copy(acc_sh.at[pl.ds(r0, zrows)],
                                out_hbm.at[cid, hd, pl.ds(r0, zrows)])
                return carry

            lax.fori_loop(0, rows_per_tile // zrows, copy_out, 0)
            plsc.subcore_barrier()

    return edge_pass


def _dense1_body(x_ref, w_ref, as_w_ref, ad_w_ref,
                 htab_ref, as_ref, ad_ref, ms_ref, md_ref):
    i = pl.program_id(0)
    h = jnp.dot(x_ref[:], w_ref[:], preferred_element_type=jnp.float32)
    a_s = jnp.dot(h, as_w_ref[:], preferred_element_type=jnp.float32)
    a_d = jnp.dot(h, ad_w_ref[:], preferred_element_type=jnp.float32)
    as_ref[:] = a_s
    ad_ref[:] = a_d
    bms = jnp.max(a_s, axis=0)[None, :]
    bmd = jnp.max(a_d, axis=0)[None, :]

    @pl.when(i == 0)
    def _():
        ms_ref[:] = bms
        md_ref[:] = bmd

    @pl.when(i > 0)
    def _():
        ms_ref[:] = jnp.maximum(ms_ref[:], bms)
        md_ref[:] = jnp.maximum(md_ref[:], bmd)

    for vh in range(8):
        htab_ref[vh] = h[:, 64 * vh:64 * (vh + 1)]


def _dense1(x, W1, As, Ad):
    B = 1000
    return pl.pallas_call(
        _dense1_body,
        grid=(N_NODES // B,),
        in_specs=[
            pl.BlockSpec((B, 128), lambda i: (i, 0)),
            pl.BlockSpec((128, 512), lambda i: (0, 0)),
            pl.BlockSpec((512, 4), lambda i: (0, 0)),
            pl.BlockSpec((512, 4), lambda i: (0, 0)),
        ],
        out_specs=(
            pl.BlockSpec((8, B, 64), lambda i: (0, i, 0)),
            pl.BlockSpec((B, 4), lambda i: (i, 0)),
            pl.BlockSpec((B, 4), lambda i: (i, 0)),
            pl.BlockSpec((1, 4), lambda i: (0, 0)),
            pl.BlockSpec((1, 4), lambda i: (0, 0)),
        ),
        out_shape=(
            jax.ShapeDtypeStruct((8, N_NODES, 64), jnp.float32),
            jax.ShapeDtypeStruct((N_NODES, 4), jnp.float32),
            jax.ShapeDtypeStruct((N_NODES, 4), jnp.float32),
            jax.ShapeDtypeStruct((1, 4), jnp.float32),
            jax.ShapeDtypeStruct((1, 4), jnp.float32),
        ),
    )(x, W1, As, Ad)


def _dense2_body(h1_ref, wmu_ref, wls_ref, as_w_ref, ad_w_ref,
                 htab_ref, as_ref, ad_ref, ms_ref, md_ref):
    i = pl.program_id(0)
    h1 = h1_ref[:]
    hm = jnp.dot(h1, wmu_ref[:], preferred_element_type=jnp.float32)
    hl = jnp.dot(h1, wls_ref[:], preferred_element_type=jnp.float32)
    hcat = jnp.concatenate([hm, hl], axis=1)
    a_s = jnp.dot(hcat, as_w_ref[:], preferred_element_type=jnp.float32)
    a_d = jnp.dot(hcat, ad_w_ref[:], preferred_element_type=jnp.float32)
    as_ref[:] = a_s
    ad_ref[:] = a_d
    bms = jnp.max(a_s, axis=0)[None, :]
    bmd = jnp.max(a_d, axis=0)[None, :]

    @pl.when(i == 0)
    def _():
        ms_ref[:] = bms
        md_ref[:] = bmd

    @pl.when(i > 0)
    def _():
        ms_ref[:] = jnp.maximum(ms_ref[:], bms)
        md_ref[:] = jnp.maximum(md_ref[:], bmd)

    for vh in range(4):
        htab_ref[vh] = hcat[:, 64 * vh:64 * (vh + 1)]


def _dense2(h1, Wmu, Wls, As, Ad):
    B = 1000
    return pl.pallas_call(
        _dense2_body,
        grid=(N_NODES // B,),
        in_specs=[
            pl.BlockSpec((B, 512), lambda i: (i, 0)),
            pl.BlockSpec((512, 128), lambda i: (0, 0)),
            pl.BlockSpec((512, 128), lambda i: (0, 0)),
            pl.BlockSpec((256, 4), lambda i: (0, 0)),
            pl.BlockSpec((256, 4), lambda i: (0, 0)),
        ],
        out_specs=(
            pl.BlockSpec((4, B, 64), lambda i: (0, i, 0)),
            pl.BlockSpec((B, 4), lambda i: (i, 0)),
            pl.BlockSpec((B, 4), lambda i: (i, 0)),
            pl.BlockSpec((1, 4), lambda i: (0, 0)),
            pl.BlockSpec((1, 4), lambda i: (0, 0)),
        ),
        out_shape=(
            jax.ShapeDtypeStruct((4, N_NODES, 64), jnp.float32),
            jax.ShapeDtypeStruct((N_NODES, 4), jnp.float32),
            jax.ShapeDtypeStruct((N_NODES, 4), jnp.float32),
            jax.ShapeDtypeStruct((1, 4), jnp.float32),
            jax.ShapeDtypeStruct((1, 4), jnp.float32),
        ),
    )(h1, Wmu, Wls, As, Ad)


def _combine1_body(p_ref, h_ref, as_ref, ad_ref, m_ref, b_ref, out_ref):
    m = m_ref[:]
    z = as_ref[:] + ad_ref[:]
    ws = jnp.exp(jnp.where(z > 0, z, z * 0.2) - m)
    b = b_ref[:]
    for vh in range(8):
        hd = vh // 2
        hcol = h_ref[vh]
        wcol = ws[:, hd:hd + 1]
        num = (p_ref[0, vh, :, 0:64] + p_ref[1, vh, :, 0:64]
               + wcol * hcol)
        den = (p_ref[0, vh, :, 64:65] + p_ref[1, vh, :, 64:65]
               + wcol + 1e-16)
        val = num / den + b[:, 64 * vh:64 * (vh + 1)]
        out_ref[:, 64 * vh:64 * (vh + 1)] = jnp.where(
            val > 0, val, jnp.exp(val) - 1.0)


def _combine1(p, htab, a_s, a_d, m, b):
    B = 1000
    return pl.pallas_call(
        _combine1_body,
        grid=(N_NODES // B,),
        in_specs=[
            pl.BlockSpec((2, 8, B, 80), lambda i: (0, 0, i, 0)),
            pl.BlockSpec((8, B, 64), lambda i: (0, i, 0)),
            pl.BlockSpec((B, 4), lambda i: (i, 0)),
            pl.BlockSpec((B, 4), lambda i: (i, 0)),
            pl.BlockSpec((1, 4), lambda i: (0, 0)),
            pl.BlockSpec((1, 512), lambda i: (0, 0)),
        ],
        out_specs=pl.BlockSpec((B, 512), lambda i: (i, 0)),
        out_shape=jax.ShapeDtypeStruct((N_NODES, 512), jnp.float32),
    )(p, htab, a_s, a_d, m, b)


def _combine2_body(p_ref, h_ref, as_ref, ad_ref, m_ref, bmu_ref, bls_ref,
                   mu_ref, ls_ref):
    m = m_ref[:]
    z = as_ref[:] + ad_ref[:]
    ws = jnp.exp(jnp.where(z > 0, z, z * 0.2) - m)
    res = []
    for vh in range(4):
        hcol = h_ref[vh]
        wcol = ws[:, vh:vh + 1]
        num = (p_ref[0, vh, :, 0:64] + p_ref[1, vh, :, 0:64]
               + wcol * hcol)
        den = (p_ref[0, vh, :, 64:65] + p_ref[1, vh, :, 64:65]
               + wcol + 1e-16)
        res.append(num / den)
    mu_ref[:] = 0.5 * (res[0] + res[1]) + bmu_ref[:]
    ls_ref[:] = 0.5 * (res[2] + res[3]) + bls_ref[:]


def _combine2(p, htab, a_s, a_d, m, bmu, bls):
    B = 1000
    return pl.pallas_call(
        _combine2_body,
        grid=(N_NODES // B,),
        in_specs=[
            pl.BlockSpec((2, 4, B, 80), lambda i: (0, 0, i, 0)),
            pl.BlockSpec((4, B, 64), lambda i: (0, i, 0)),
            pl.BlockSpec((B, 4), lambda i: (i, 0)),
            pl.BlockSpec((B, 4), lambda i: (i, 0)),
            pl.BlockSpec((1, 4), lambda i: (0, 0)),
            pl.BlockSpec((1, 64), lambda i: (0, 0)),
            pl.BlockSpec((1, 64), lambda i: (0, 0)),
        ],
        out_specs=(
            pl.BlockSpec((B, 64), lambda i: (i, 0)),
            pl.BlockSpec((B, 64), lambda i: (i, 0)),
        ),
        out_shape=(
            jax.ShapeDtypeStruct((N_NODES, 64), jnp.float32),
            jax.ShapeDtypeStruct((N_NODES, 64), jnp.float32),
        ),
    )(p, htab, a_s, a_d, m, bmu, bls)


def _block_diag_att(att, heads, ch):
    eye = jnp.eye(heads, dtype=jnp.float32)
    return (att[0][:, :, None] * eye[:, None, :]).reshape(heads * ch, heads)


def kernel(x, edge_index, W1, a_src1, a_dst1, b1,
           W_mu, a_src_mu, a_dst_mu, b_mu, W_ls, a_src_ls, a_dst_ls, b_ls):
    E = edge_index.shape[1]
    src = edge_index[0]
    dst = edge_index[1]

    edge64x8 = _make_edge_pass(E, 64, 8)
    edge64 = _make_edge_pass(E, 64, 4)

    As1 = _block_diag_att(a_src1, 4, 128)
    Ad1 = _block_diag_att(a_dst1, 4, 128)
    htab1, a_s1, a_d1, ms1, md1 = _dense1(x, W1, As1, Ad1)
    m1 = jnp.maximum(ms1 + md1, 0.0)
    m1_16 = jnp.broadcast_to(
        jnp.repeat(m1.reshape(4, 1), 2, axis=0), (8, 16))
    as1_flat = jnp.repeat(a_s1.T, 2, axis=0).reshape(-1)
    ad1_flat = jnp.repeat(a_d1.T, 2, axis=0).reshape(-1)
    p1 = edge64x8(src, dst, htab1.reshape(8 * N_NODES, 64),
                  as1_flat, ad1_flat, m1_16)
    h1 = _combine1(p1, htab1, a_s1, a_d1, m1, b1.reshape(1, 512))

    acat_s = jnp.concatenate([a_src_mu[0], a_src_ls[0]], axis=0)
    acat_d = jnp.concatenate([a_dst_mu[0], a_dst_ls[0]], axis=0)
    eye4 = jnp.eye(4, dtype=jnp.float32)
    As2 = (acat_s[:, :, None] * eye4[:, None, :]).reshape(256, 4)
    Ad2 = (acat_d[:, :, None] * eye4[:, None, :]).reshape(256, 4)
    htab2, a_s2, a_d2, ms2, md2 = _dense2(h1, W_mu, W_ls, As2, Ad2)
    m2 = jnp.maximum(ms2 + md2, 0.0)
    m2_16 = jnp.broadcast_to(m2.reshape(4, 1), (4, 16))
    as2_flat = a_s2.T.reshape(-1)
    ad2_flat = a_d2.T.reshape(-1)
    p2 = edge64(src, dst, htab2.reshape(4 * N_NODES, 64),
                as2_flat, ad2_flat, m2_16)
    mu, logstd = _combine2(p2, htab2, a_s2, a_d2, m2,
                           b_mu.reshape(1, 64), b_ls.reshape(1, 64))
    return (mu, logstd)


# speedup vs baseline: 28.6533x; 1.9803x over previous
import functools

import jax
import jax.numpy as jnp
from jax import lax
from jax.experimental import pallas as pl
from jax.experimental.pallas import tpu as pltpu
from jax.experimental.pallas import tpu_sc as plsc

N_NODES = 10000
N_TILES = 32


def _make_edge_pass(E, C, heads):
    OUTW = C + 16
    e_per_tile = E // N_TILES
    K = 80
    nchunk = e_per_tile // K
    npair = nchunk // 2
    accn = 10240
    rows_per_tile = accn // 16
    zrows = 128
    mesh = plsc.VectorSubcoreMesh(core_axis_name="c", subcore_axis_name="s")

    @functools.partial(
        pl.kernel,
        mesh=mesh,
        compiler_params=pltpu.CompilerParams(
            needs_layout_passes=False, use_tc_tiling_on_sc=False),
        out_type=jax.ShapeDtypeStruct((2, heads, accn, OUTW), jnp.float32),
        scratch_types=[
            pltpu.VMEM((e_per_tile,), jnp.int32),
            pltpu.VMEM((e_per_tile,), jnp.int32),
            pltpu.VMEM((N_NODES,), jnp.float32),
            pltpu.VMEM((N_NODES,), jnp.float32),
            pltpu.VMEM((heads, 16), jnp.float32),
            pltpu.VMEM((2, K), jnp.int32),
            pltpu.VMEM((2, K), jnp.int32),
            pltpu.VMEM((2, K), jnp.int32),
            pltpu.VMEM((2, K), jnp.float32),
            pltpu.VMEM((2, K, C), jnp.float32),
            pltpu.VMEM((2, K, OUTW), jnp.float32),
            pltpu.VMEM((zrows, OUTW), jnp.float32),
            pltpu.VMEM_SHARED((accn, OUTW), jnp.float32),
            pltpu.SemaphoreType.DMA,
            pltpu.SemaphoreType.DMA,
            pltpu.SemaphoreType.DMA,
            pltpu.SemaphoreType.DMA,
        ],
    )
    def edge_pass(src_hbm, dst_hbm, htab_hbm, as_hbm, ad_hbm, m_hbm, out_hbm,
                  src_all, dst_all, as_v, ad_v, m_v,
                  srcp_r, dst_r, dsts_r, w_r, rin_r, rout_r, zero_v, acc_sh,
                  semga, semgb, semsa, semsb):
        cid = lax.axis_index("c")
        sid = lax.axis_index("s")
        zvec = jnp.zeros((16,), jnp.float32)
        lane0 = jnp.arange(16, dtype=jnp.int32) == 0

        def zfill(i, carry):
            for j in range(OUTW // 16):
                zero_v[i, pl.ds(j * 16, 16)] = zvec
            return carry

        lax.fori_loop(0, zrows, zfill, 0)
        zvec_i = jnp.zeros((16,), jnp.int32)
        for r in range(2):
            for j in range(K // 16):
                dsts_r[r, pl.ds(j * 16, 16)] = zvec_i
        pltpu.sync_copy(m_hbm, m_v)
        row_base = sid * rows_per_tile
        ebase = cid * (E // 2) + sid * e_per_tile
        pltpu.sync_copy(src_hbm.at[pl.ds(ebase, e_per_tile)], src_all)
        pltpu.sync_copy(dst_hbm.at[pl.ds(ebase, e_per_tile)], dst_all)

        def head_body(hd, hcarry):
            pltpu.sync_copy(as_hbm.at[pl.ds(hd * N_NODES, N_NODES)], as_v)
            pltpu.sync_copy(ad_hbm.at[pl.ds(hd * N_NODES, N_NODES)], ad_v)

            def zero_acc(i, carry):
                pltpu.sync_copy(zero_v,
                                acc_sh.at[pl.ds(row_base + i * zrows, zrows)])
                return carry

            lax.fori_loop(0, rows_per_tile // zrows, zero_acc, 0)
            plsc.subcore_barrier()

            mvec = m_v[hd, :]

            def build(c, r):
                off = c * K
                for j in range(K // 16):
                    s16 = src_all[pl.ds(off + j * 16, 16)]
                    d16 = dst_all[pl.ds(off + j * 16, 16)]
                    srcp_r[r, pl.ds(j * 16, 16)] = s16 + hd * N_NODES
                    dst_r[r, pl.ds(j * 16, 16)] = d16
                    z = (plsc.load_gather(as_v, [s16])
                         + plsc.load_gather(ad_v, [d16]))
                    alpha = jnp.where(z > 0, z, z * 0.2)
                    w_r[r, pl.ds(j * 16, 16)] = jnp.exp(alpha - mvec)

            def start_gather(r, sem):
                pltpu.async_copy(htab_hbm.at[srcp_r.at[r]], rin_r.at[r], sem)

            def wait_gather(r, sem):
                pltpu.make_async_copy(
                    htab_hbm.at[srcp_r.at[r]], rin_r.at[r], sem).wait()

            def compute(r):
                def grp(g, c2):
                    w16 = w_r[r, pl.ds(g * 16, 16)]
                    for l in range(16):
                        ws = w16[l]
                        e = g * 16 + l
                        for q in range(C // 16):
                            rout_r[r, e, pl.ds(q * 16, 16)] = (
                                rin_r[r, e, pl.ds(q * 16, 16)] * ws)
                        rout_r[r, e, pl.ds(C, 16)] = jnp.where(lane0, ws, 0.0)
                    return c2

                lax.fori_loop(0, K // 16, grp, 0)
                for j in range(K // 16):
                    dsts_r[r, pl.ds(j * 16, 16)] = dst_r[r, pl.ds(j * 16, 16)]

            def start_scatter(r, sem):
                pltpu.async_copy(rout_r.at[r], acc_sh.at[dsts_r.at[r]], sem,
                                 add=True)

            def wait_scatter(r, sem):
                pltpu.make_async_copy(
                    rout_r.at[r], acc_sh.at[dsts_r.at[r]], sem).wait()

            pltpu.async_copy(zero_v.at[pl.ds(0, K)], acc_sh.at[dsts_r.at[0]],
                             semsa, add=True)
            pltpu.async_copy(zero_v.at[pl.ds(0, K)], acc_sh.at[dsts_r.at[1]],
                             semsb, add=True)
            build(0, 0)
            start_gather(0, semga)
            build(1, 1)
            start_gather(1, semgb)

            def pair(i, carry):
                wait_gather(0, semga)
                wait_scatter(0, semsa)
                compute(0)
                start_scatter(0, semsa)
                build(2 * i + 2, 0)
                start_gather(0, semga)
                wait_gather(1, semgb)
                wait_scatter(1, semsb)
                compute(1)
                start_scatter(1, semsb)

                @pl.when(i < npair - 1)
                def _():
                    build(2 * i + 3, 1)
                    start_gather(1, semgb)

                return carry

            lax.fori_loop(0, npair, pair, 0)
            wait_gather(0, semga)
            wait_scatter(0, semsa)
            compute(0)
            start_scatter(0, semsa)
            wait_scatter(0, semsa)
            wait_scatter(1, semsb)
            plsc.subcore_barrier()

            def copy_out(i, carry):
                r0 = row_base + i * zrows
                pltpu.sync_copy(acc_sh.at[pl.ds(r0, zrows)],
                                out_hbm.at[cid, hd, pl.ds(r0, zrows)])
                return carry

            lax.fori_loop(0, rows_per_tile // zrows, copy_out, 0)
            plsc.subcore_barrier()
            return hcarry

        lax.fori_loop(0, heads, head_body, 0)

    return edge_pass


def _dense1_body(x_ref, w_ref, as_w_ref, ad_w_ref,
                 htab_ref, as_ref, ad_ref, ms_ref, md_ref):
    i = pl.program_id(0)
    h = jnp.dot(x_ref[:], w_ref[:], preferred_element_type=jnp.float32)
    a_s = jnp.dot(h, as_w_ref[:], preferred_element_type=jnp.float32)
    a_d = jnp.dot(h, ad_w_ref[:], preferred_element_type=jnp.float32)
    as_ref[:] = a_s
    ad_ref[:] = a_d
    bms = jnp.max(a_s, axis=0)[None, :]
    bmd = jnp.max(a_d, axis=0)[None, :]

    @pl.when(i == 0)
    def _():
        ms_ref[:] = bms
        md_ref[:] = bmd

    @pl.when(i > 0)
    def _():
        ms_ref[:] = jnp.maximum(ms_ref[:], bms)
        md_ref[:] = jnp.maximum(md_ref[:], bmd)

    for vh in range(8):
        htab_ref[vh] = h[:, 64 * vh:64 * (vh + 1)]


def _dense1(x, W1, As, Ad):
    B = 1000
    return pl.pallas_call(
        _dense1_body,
        grid=(N_NODES // B,),
        in_specs=[
            pl.BlockSpec((B, 128), lambda i: (i, 0)),
            pl.BlockSpec((128, 512), lambda i: (0, 0)),
            pl.BlockSpec((512, 4), lambda i: (0, 0)),
            pl.BlockSpec((512, 4), lambda i: (0, 0)),
        ],
        out_specs=(
            pl.BlockSpec((8, B, 64), lambda i: (0, i, 0)),
            pl.BlockSpec((B, 4), lambda i: (i, 0)),
            pl.BlockSpec((B, 4), lambda i: (i, 0)),
            pl.BlockSpec((1, 4), lambda i: (0, 0)),
            pl.BlockSpec((1, 4), lambda i: (0, 0)),
        ),
        out_shape=(
            jax.ShapeDtypeStruct((8, N_NODES, 64), jnp.float32),
            jax.ShapeDtypeStruct((N_NODES, 4), jnp.float32),
            jax.ShapeDtypeStruct((N_NODES, 4), jnp.float32),
            jax.ShapeDtypeStruct((1, 4), jnp.float32),
            jax.ShapeDtypeStruct((1, 4), jnp.float32),
        ),
    )(x, W1, As, Ad)


def _dense2_body(h1_ref, wmu_ref, wls_ref, as_w_ref, ad_w_ref,
                 htab_ref, as_ref, ad_ref, ms_ref, md_ref):
    i = pl.program_id(0)
    h1 = h1_ref[:]
    hm = jnp.dot(h1, wmu_ref[:], preferred_element_type=jnp.float32)
    hl = jnp.dot(h1, wls_ref[:], preferred_element_type=jnp.float32)
    hcat = jnp.concatenate([hm, hl], axis=1)
    a_s = jnp.dot(hcat, as_w_ref[:], preferred_element_type=jnp.float32)
    a_d = jnp.dot(hcat, ad_w_ref[:], preferred_element_type=jnp.float32)
    as_ref[:] = a_s
    ad_ref[:] = a_d
    bms = jnp.max(a_s, axis=0)[None, :]
    bmd = jnp.max(a_d, axis=0)[None, :]

    @pl.when(i == 0)
    def _():
        ms_ref[:] = bms
        md_ref[:] = bmd

    @pl.when(i > 0)
    def _():
        ms_ref[:] = jnp.maximum(ms_ref[:], bms)
        md_ref[:] = jnp.maximum(md_ref[:], bmd)

    for vh in range(4):
        htab_ref[vh] = hcat[:, 64 * vh:64 * (vh + 1)]


def _dense2(h1, Wmu, Wls, As, Ad):
    B = 1000
    return pl.pallas_call(
        _dense2_body,
        grid=(N_NODES // B,),
        in_specs=[
            pl.BlockSpec((B, 512), lambda i: (i, 0)),
            pl.BlockSpec((512, 128), lambda i: (0, 0)),
            pl.BlockSpec((512, 128), lambda i: (0, 0)),
            pl.BlockSpec((256, 4), lambda i: (0, 0)),
            pl.BlockSpec((256, 4), lambda i: (0, 0)),
        ],
        out_specs=(
            pl.BlockSpec((4, B, 64), lambda i: (0, i, 0)),
            pl.BlockSpec((B, 4), lambda i: (i, 0)),
            pl.BlockSpec((B, 4), lambda i: (i, 0)),
            pl.BlockSpec((1, 4), lambda i: (0, 0)),
            pl.BlockSpec((1, 4), lambda i: (0, 0)),
        ),
        out_shape=(
            jax.ShapeDtypeStruct((4, N_NODES, 64), jnp.float32),
            jax.ShapeDtypeStruct((N_NODES, 4), jnp.float32),
            jax.ShapeDtypeStruct((N_NODES, 4), jnp.float32),
            jax.ShapeDtypeStruct((1, 4), jnp.float32),
            jax.ShapeDtypeStruct((1, 4), jnp.float32),
        ),
    )(h1, Wmu, Wls, As, Ad)


def _combine1_body(p_ref, h_ref, as_ref, ad_ref, m_ref, b_ref, out_ref):
    m = m_ref[:]
    z = as_ref[:] + ad_ref[:]
    ws = jnp.exp(jnp.where(z > 0, z, z * 0.2) - m)
    b = b_ref[:]
    for vh in range(8):
        hd = vh // 2
        hcol = h_ref[vh]
        wcol = ws[:, hd:hd + 1]
        num = (p_ref[0, vh, :, 0:64] + p_ref[1, vh, :, 0:64]
               + wcol * hcol)
        den = (p_ref[0, vh, :, 64:65] + p_ref[1, vh, :, 64:65]
               + wcol + 1e-16)
        val = num / den + b[:, 64 * vh:64 * (vh + 1)]
        out_ref[:, 64 * vh:64 * (vh + 1)] = jnp.where(
            val > 0, val, jnp.exp(val) - 1.0)


def _combine1(p, htab, a_s, a_d, m, b):
    B = 1000
    return pl.pallas_call(
        _combine1_body,
        grid=(N_NODES // B,),
        in_specs=[
            pl.BlockSpec((2, 8, B, 80), lambda i: (0, 0, i, 0)),
            pl.BlockSpec((8, B, 64), lambda i: (0, i, 0)),
            pl.BlockSpec((B, 4), lambda i: (i, 0)),
            pl.BlockSpec((B, 4), lambda i: (i, 0)),
            pl.BlockSpec((1, 4), lambda i: (0, 0)),
            pl.BlockSpec((1, 512), lambda i: (0, 0)),
        ],
        out_specs=pl.BlockSpec((B, 512), lambda i: (i, 0)),
        out_shape=jax.ShapeDtypeStruct((N_NODES, 512), jnp.float32),
    )(p, htab, a_s, a_d, m, b)


def _combine2_body(p_ref, h_ref, as_ref, ad_ref, m_ref, bmu_ref, bls_ref,
                   mu_ref, ls_ref):
    m = m_ref[:]
    z = as_ref[:] + ad_ref[:]
    ws = jnp.exp(jnp.where(z > 0, z, z * 0.2) - m)
    res = []
    for vh in range(4):
        hcol = h_ref[vh]
        wcol = ws[:, vh:vh + 1]
        num = (p_ref[0, vh, :, 0:64] + p_ref[1, vh, :, 0:64]
               + wcol * hcol)
        den = (p_ref[0, vh, :, 64:65] + p_ref[1, vh, :, 64:65]
               + wcol + 1e-16)
        res.append(num / den)
    mu_ref[:] = 0.5 * (res[0] + res[1]) + bmu_ref[:]
    ls_ref[:] = 0.5 * (res[2] + res[3]) + bls_ref[:]


def _combine2(p, htab, a_s, a_d, m, bmu, bls):
    B = 1000
    return pl.pallas_call(
        _combine2_body,
        grid=(N_NODES // B,),
        in_specs=[
            pl.BlockSpec((2, 4, B, 80), lambda i: (0, 0, i, 0)),
            pl.BlockSpec((4, B, 64), lambda i: (0, i, 0)),
            pl.BlockSpec((B, 4), lambda i: (i, 0)),
            pl.BlockSpec((B, 4), lambda i: (i, 0)),
            pl.BlockSpec((1, 4), lambda i: (0, 0)),
            pl.BlockSpec((1, 64), lambda i: (0, 0)),
            pl.BlockSpec((1, 64), lambda i: (0, 0)),
        ],
        out_specs=(
            pl.BlockSpec((B, 64), lambda i: (i, 0)),
            pl.BlockSpec((B, 64), lambda i: (i, 0)),
        ),
        out_shape=(
            jax.ShapeDtypeStruct((N_NODES, 64), jnp.float32),
            jax.ShapeDtypeStruct((N_NODES, 64), jnp.float32),
        ),
    )(p, htab, a_s, a_d, m, bmu, bls)


def _block_diag_att(att, heads, ch):
    eye = jnp.eye(heads, dtype=jnp.float32)
    return (att[0][:, :, None] * eye[:, None, :]).reshape(heads * ch, heads)


def kernel(x, edge_index, W1, a_src1, a_dst1, b1,
           W_mu, a_src_mu, a_dst_mu, b_mu, W_ls, a_src_ls, a_dst_ls, b_ls):
    E = edge_index.shape[1]
    src = edge_index[0]
    dst = edge_index[1]

    edge64x8 = _make_edge_pass(E, 64, 8)
    edge64 = _make_edge_pass(E, 64, 4)

    As1 = _block_diag_att(a_src1, 4, 128)
    Ad1 = _block_diag_att(a_dst1, 4, 128)
    htab1, a_s1, a_d1, ms1, md1 = _dense1(x, W1, As1, Ad1)
    m1 = jnp.maximum(ms1 + md1, 0.0)
    m1_16 = jnp.broadcast_to(
        jnp.repeat(m1.reshape(4, 1), 2, axis=0), (8, 16))
    as1_flat = jnp.repeat(a_s1.T, 2, axis=0).reshape(-1)
    ad1_flat = jnp.repeat(a_d1.T, 2, axis=0).reshape(-1)
    p1 = edge64x8(src, dst, htab1.reshape(8 * N_NODES, 64),
                  as1_flat, ad1_flat, m1_16)
    h1 = _combine1(p1, htab1, a_s1, a_d1, m1, b1.reshape(1, 512))

    acat_s = jnp.concatenate([a_src_mu[0], a_src_ls[0]], axis=0)
    acat_d = jnp.concatenate([a_dst_mu[0], a_dst_ls[0]], axis=0)
    eye4 = jnp.eye(4, dtype=jnp.float32)
    As2 = (acat_s[:, :, None] * eye4[:, None, :]).reshape(256, 4)
    Ad2 = (acat_d[:, :, None] * eye4[:, None, :]).reshape(256, 4)
    htab2, a_s2, a_d2, ms2, md2 = _dense2(h1, W_mu, W_ls, As2, Ad2)
    m2 = jnp.maximum(ms2 + md2, 0.0)
    m2_16 = jnp.broadcast_to(m2.reshape(4, 1), (4, 16))
    as2_flat = a_s2.T.reshape(-1)
    ad2_flat = a_d2.T.reshape(-1)
    p2 = edge64(src, dst, htab2.reshape(4 * N_NODES, 64),
                as2_flat, ad2_flat, m2_16)
    mu, logstd = _combine2(p2, htab2, a_s2, a_d2, m2,
                           b_mu.reshape(1, 64), b_ls.reshape(1, 64))
    return (mu, logstd)
